# baseline (speedup 1.0000x reference)
import jax
import jax.numpy as jnp
from jax.experimental import pallas as pl
from jax.experimental.pallas import tpu as pltpu


def _body(ts_ref, logits_ref, boxes_ref, cls_ref,
          scores_ref, labels_ref, boxes_out_ref, cls_out_ref):
    i = pl.program_id(0)

    xt = logits_ref[0].T
    c_iota = jax.lax.broadcasted_iota(jnp.int32, xt.shape, 0)
    m = jnp.max(xt, axis=0)
    labels = jnp.min(jnp.where(xt == m[None, :], c_iota, 91), axis=0)
    s = jnp.sum(jnp.exp(xt), axis=0)
    scores_ref[i, :] = jnp.exp(m) / s
    labels_ref[i, :] = labels

    bt = boxes_ref[0].T
    cx, cy, w, h = bt[0], bt[1], bt[2], bt[3]
    sh = ts_ref[i, 0].astype(jnp.float32)
    sw = ts_ref[i, 1].astype(jnp.float32)
    x0 = jnp.clip(cx - 0.5 * w, 0.0, 1.0) * sw
    y0 = jnp.clip(cy - 0.5 * h, 0.0, 1.0) * sh
    x1 = jnp.clip(cx + 0.5 * w, 0.0, 1.0) * sw
    y1 = jnp.clip(cy + 0.5 * h, 0.0, 1.0) * sh
    boxes_out_ref[i] = jnp.stack([x0, y0, x1, y1], axis=0).T

    @pl.when(i == 0)
    def _():
        c = cls_ref[...]
        cm = jnp.max(c, axis=-1, keepdims=True)
        ci = jax.lax.broadcasted_iota(jnp.int32, c.shape, 1)
        cls_out_ref[...] = jnp.min(jnp.where(c == cm, ci, 10),
                                   axis=-1, keepdims=True)


def kernel(pred_logits, pred_boxes, cls_logits, target_sizes):
    nb, nq, nc = pred_logits.shape
    scores, labels, boxes, cls2 = pl.pallas_call(
        _body,
        grid=(nb,),
        in_specs=[
            pl.BlockSpec(memory_space=pltpu.SMEM),
            pl.BlockSpec((1, nq, nc), lambda i: (i, 0, 0)),
            pl.BlockSpec((1, nq, 4), lambda i: (i, 0, 0)),
            pl.BlockSpec((16, 10), lambda i: (0, 0)),
        ],
        out_specs=[
            pl.BlockSpec((nb, nq), lambda i: (0, 0)),
            pl.BlockSpec((nb, nq), lambda i: (0, 0)),
            pl.BlockSpec((nb, nq, 4), lambda i: (0, 0, 0)),
            pl.BlockSpec((16, 1), lambda i: (0, 0)),
        ],
        out_shape=[
            jax.ShapeDtypeStruct((nb, nq), jnp.float32),
            jax.ShapeDtypeStruct((nb, nq), jnp.int32),
            jax.ShapeDtypeStruct((nb, nq, 4), jnp.float32),
            jax.ShapeDtypeStruct((16, 1), jnp.int32),
        ],
    )(target_sizes, pred_logits, pred_boxes, cls_logits)
    return scores, labels, boxes, cls2.reshape(nb)
